# Initial kernel scaffold; baseline (speedup 1.0000x reference)
import jax
import jax.numpy as jnp
from jax.experimental import pallas as pl


def kernel(x, edge_index, batch_size):
    raise NotImplementedError("write your pallas kernel here")


# speedup vs baseline: 5.5475x; 5.5475x over previous
import functools

import jax
import jax.numpy as jnp
from jax import lax
from jax.experimental import pallas as pl
from jax.experimental.pallas import tpu as pltpu
from jax.experimental.pallas import tpu_sc as plsc

NC = 2
NS = 16
NW = NC * NS

K = 80


def _sc_scatter_add(x, src, dst):
    n_nodes, d_feat = x.shape
    n_edges = src.shape[0]
    e_per_w = n_edges // NW
    n_chunks = e_per_w // K
    n_pad = ((n_nodes + NS * 8 - 1) // (NS * 8)) * (NS * 8)
    rows_per_tile = n_pad // NS
    zreps = 4
    zr = rows_per_tile // zreps

    mesh = plsc.VectorSubcoreMesh(
        core_axis_name="c", subcore_axis_name="s", num_cores=NC, num_subcores=NS
    )

    @functools.partial(
        pl.kernel,
        out_type=jax.ShapeDtypeStruct((NC, n_pad, d_feat), jnp.float32),
        mesh=mesh,
        scratch_types=[
            pltpu.VMEM((K,), jnp.int32),
            pltpu.VMEM((K,), jnp.int32),
            pltpu.VMEM((K, d_feat), jnp.float32),
            pltpu.VMEM((zr, d_feat), jnp.float32),
            pltpu.VMEM_SHARED((n_pad, d_feat), jnp.float32),
            pltpu.SemaphoreType.DMA,
        ],
    )
    def sc_kernel(x_hbm, src_hbm, dst_hbm, out_hbm, src_v, dst_v, rows_v, zbuf, acc, sem):
        c = lax.axis_index("c")
        s = lax.axis_index("s")
        wid = s * NC + c

        def zero_row(i, carry):
            for j in range(d_feat // 16):
                zbuf[i, pl.ds(j * 16, 16)] = jnp.zeros((16,), jnp.float32)
            return carry

        lax.fori_loop(0, zr, zero_row, 0)
        row_base = s * rows_per_tile
        for b in range(zreps):
            pltpu.sync_copy(zbuf, acc.at[pl.ds(row_base + b * zr, zr)])
        plsc.subcore_barrier()

        edge_base = wid * e_per_w

        def chunk(g, carry):
            off = edge_base + g * K
            pltpu.sync_copy(src_hbm.at[pl.ds(off, K)], src_v)
            pltpu.sync_copy(dst_hbm.at[pl.ds(off, K)], dst_v)
            pltpu.async_copy(x_hbm.at[src_v], rows_v, sem).wait()
            pltpu.sync_copy(rows_v, acc.at[dst_v], add=True)
            return carry

        lax.fori_loop(0, n_chunks, chunk, 0)
        plsc.subcore_barrier()

        pltpu.sync_copy(
            acc.at[pl.ds(row_base, rows_per_tile)],
            out_hbm.at[c, pl.ds(row_base, rows_per_tile)],
        )

    return sc_kernel(x, src, dst)


def _finalize(x, acc):
    n_nodes, d_feat = x.shape
    br = 1000

    def body(x_ref, acc_ref, o_ref):
        o_ref[:, :d_feat] = x_ref[...]
        o_ref[:, d_feat:] = acc_ref[0] + acc_ref[1]

    return pl.pallas_call(
        body,
        grid=(n_nodes // br,),
        in_specs=[
            pl.BlockSpec((br, d_feat), lambda i: (i, 0)),
            pl.BlockSpec((NC, br, d_feat), lambda i: (0, i, 0)),
        ],
        out_specs=pl.BlockSpec((br, 2 * d_feat), lambda i: (i, 0)),
        out_shape=jax.ShapeDtypeStruct((n_nodes, 2 * d_feat), jnp.float32),
    )(x, acc)


@jax.jit
def _run(x, src, dst):
    acc = _sc_scatter_add(x, src, dst)
    return _finalize(x, acc)


def kernel(x, edge_index, batch_size):
    src = edge_index[0].astype(jnp.int32)
    dst = edge_index[1].astype(jnp.int32)
    return _run(x, src, dst)


# speedup vs baseline: 6.8016x; 1.2261x over previous
import functools

import jax
import jax.numpy as jnp
from jax import lax
from jax.experimental import pallas as pl
from jax.experimental.pallas import tpu as pltpu
from jax.experimental.pallas import tpu_sc as plsc

NC = 2
NS = 16
NW = NC * NS

K = 80
NBUF = 3
NIDX = 6


def _sc_scatter_add(x, edges, n_chunks):
    n_nodes, d_feat = x.shape
    n_pad = ((n_nodes + NS * 8 - 1) // (NS * 8)) * (NS * 8)
    rows_per_tile = n_pad // NS

    mesh = plsc.VectorSubcoreMesh(
        core_axis_name="c", subcore_axis_name="s", num_cores=NC, num_subcores=NS
    )

    @functools.partial(
        pl.kernel,
        out_type=jax.ShapeDtypeStruct((NC, n_pad, d_feat), jnp.float32),
        mesh=mesh,
        scratch_types=[
            pltpu.VMEM((NIDX, 2, K), jnp.int32),
            pltpu.VMEM((NBUF, K, d_feat), jnp.float32),
            pltpu.VMEM_SHARED((n_pad, d_feat), jnp.float32),
            [pltpu.SemaphoreType.DMA] * NIDX,
            [pltpu.SemaphoreType.DMA] * NBUF,
        ],
    )
    def sc_kernel(x_hbm, e_hbm, out_hbm, idx_v, rows_v, acc, isems, gsems):
        c = lax.axis_index("c")
        s = lax.axis_index("s")
        wid = s * NC + c

        def zero_row(i, carry):
            for j in range(d_feat // 16):
                rows_v[0, i, pl.ds(j * 16, 16)] = jnp.zeros((16,), jnp.float32)
            return carry

        lax.fori_loop(0, K, zero_row, 0)
        row_base = s * rows_per_tile
        nfull = rows_per_tile // K
        for b in range(nfull):
            pltpu.sync_copy(rows_v.at[0], acc.at[pl.ds(row_base + b * K, K)])
        rem = rows_per_tile - nfull * K
        if rem:
            pltpu.sync_copy(
                rows_v.at[0, pl.ds(0, rem)],
                acc.at[pl.ds(row_base + nfull * K, rem)],
            )
        plsc.subcore_barrier()

        def idx_start(g, i):
            pltpu.async_copy(e_hbm.at[wid, g], idx_v.at[i], isems[i])

        def idx_wait(i):
            pltpu.make_async_copy(e_hbm.at[wid, 0], idx_v.at[i], isems[i]).wait()

        def gather_start(g, i, b):
            pltpu.async_copy(x_hbm.at[idx_v.at[i, 0]], rows_v.at[b], gsems[b])

        def gather_wait(b):
            pltpu.make_async_copy(
                x_hbm.at[idx_v.at[0, 0]], rows_v.at[b], gsems[b]
            ).wait()

        def scatter(i, b):
            pltpu.sync_copy(rows_v.at[b], acc.at[idx_v.at[i, 1]], add=True)

        for g in range(NIDX):
            idx_start(g, g)
        for g in range(NBUF):
            idx_wait(g)
            gather_start(g, g, g)

        @pl.loop(0, n_chunks - NIDX, step=NIDX)
        def _(g0):
            for b in range(NIDX):
                g = g0 + b
                gather_wait(b % NBUF)
                scatter(b, b % NBUF)
                idx_start(g + NIDX, b)
                idx_wait((b + NBUF) % NIDX)
                gather_start(g + NBUF, (b + NBUF) % NIDX, b % NBUF)

        for b in range(NIDX):
            g = n_chunks - NIDX + b
            gather_wait(b % NBUF)
            scatter(b, b % NBUF)
            if b + NBUF < NIDX:
                idx_wait((b + NBUF) % NIDX)
                gather_start(g + NBUF, (b + NBUF) % NIDX, b % NBUF)
        plsc.subcore_barrier()

        pltpu.sync_copy(
            acc.at[pl.ds(row_base, rows_per_tile)],
            out_hbm.at[c, pl.ds(row_base, rows_per_tile)],
        )

    return sc_kernel(x, edges)


def _finalize(x, acc):
    n_nodes, d_feat = x.shape
    br = 1000

    def body(x_ref, acc_ref, o_ref):
        o_ref[:, :d_feat] = x_ref[...]
        o_ref[:, d_feat:] = acc_ref[0] + acc_ref[1]

    return pl.pallas_call(
        body,
        grid=(n_nodes // br,),
        in_specs=[
            pl.BlockSpec((br, d_feat), lambda i: (i, 0)),
            pl.BlockSpec((NC, br, d_feat), lambda i: (0, i, 0)),
        ],
        out_specs=pl.BlockSpec((br, 2 * d_feat), lambda i: (i, 0)),
        out_shape=jax.ShapeDtypeStruct((n_nodes, 2 * d_feat), jnp.float32),
    )(x, acc)


@jax.jit
def _run(x, edges):
    n_chunks = edges.shape[1]
    acc = _sc_scatter_add(x, edges, n_chunks)
    return _finalize(x, acc)


def kernel(x, edge_index, batch_size):
    n_nodes = x.shape[0]
    n_edges = edge_index.shape[1]
    ei = edge_index.astype(jnp.int32)
    quantum = NW * NIDX * K
    n_padded = ((n_edges + quantum - 1) // quantum) * quantum
    pad = n_padded - n_edges
    src = jnp.concatenate([ei[0], jnp.zeros((pad,), jnp.int32)])
    dst = jnp.concatenate([ei[1], jnp.full((pad,), n_nodes, jnp.int32)])
    edges = jnp.stack(
        [src.reshape(NW, -1, K), dst.reshape(NW, -1, K)], axis=2
    )
    return _run(x, edges)


# speedup vs baseline: 7.7043x; 1.1327x over previous
import functools

import jax
import jax.numpy as jnp
from jax import lax
from jax.experimental import pallas as pl
from jax.experimental.pallas import tpu as pltpu
from jax.experimental.pallas import tpu_sc as plsc

NC = 2
NS = 16
NW = NC * NS

K = 80
NBUF = 3
NIDX = 6
NCH0 = 174
NCH1 = 78


def _sc_scatter_add(x, edges):
    n_nodes, d_feat = x.shape
    n_pad = ((n_nodes + NS * 8 - 1) // (NS * 8)) * (NS * 8)
    rows_per_tile = n_pad // NS

    mesh = plsc.VectorSubcoreMesh(
        core_axis_name="c", subcore_axis_name="s", num_cores=NC, num_subcores=NS
    )

    @functools.partial(
        pl.kernel,
        out_type=jax.ShapeDtypeStruct((NC, n_pad, d_feat), jnp.float32),
        mesh=mesh,
        scratch_types=[
            pltpu.VMEM((NIDX, 2, K), jnp.int32),
            pltpu.VMEM((NBUF, K, d_feat), jnp.float32),
            pltpu.VMEM_SHARED((n_pad, d_feat), jnp.float32),
            [pltpu.SemaphoreType.DMA] * NIDX,
            [pltpu.SemaphoreType.DMA] * NBUF,
        ],
    )
    def sc_kernel(x_hbm, e_hbm, out_hbm, idx_v, rows_v, acc, isems, gsems):
        c = lax.axis_index("c")
        s = lax.axis_index("s")
        n_chunks = jnp.where(c == 0, NCH0, NCH1)
        chunk_base = jnp.where(c == 0, s * NCH0, NS * NCH0 + s * NCH1)

        def zero_row(i, carry):
            for j in range(d_feat // 16):
                rows_v[0, i, pl.ds(j * 16, 16)] = jnp.zeros((16,), jnp.float32)
            return carry

        lax.fori_loop(0, K, zero_row, 0)
        row_base = s * rows_per_tile
        nfull = rows_per_tile // K
        for b in range(nfull):
            pltpu.sync_copy(rows_v.at[0], acc.at[pl.ds(row_base + b * K, K)])
        rem = rows_per_tile - nfull * K
        if rem:
            pltpu.sync_copy(
                rows_v.at[0, pl.ds(0, rem)],
                acc.at[pl.ds(row_base + nfull * K, rem)],
            )
        plsc.subcore_barrier()

        def idx_start(g, i):
            pltpu.async_copy(e_hbm.at[chunk_base + g], idx_v.at[i], isems[i])

        def idx_wait(i):
            pltpu.make_async_copy(e_hbm.at[0], idx_v.at[i], isems[i]).wait()

        def gather_start(g, i, b):
            pltpu.async_copy(x_hbm.at[idx_v.at[i, 0]], rows_v.at[b], gsems[b])

        def gather_wait(b):
            pltpu.make_async_copy(
                x_hbm.at[idx_v.at[0, 0]], rows_v.at[b], gsems[b]
            ).wait()

        def scatter(i, b):
            pltpu.sync_copy(rows_v.at[b], acc.at[idx_v.at[i, 1]], add=True)

        for g in range(NIDX):
            idx_start(g, g)
        for g in range(NBUF):
            idx_wait(g)
            gather_start(g, g, g)

        @pl.loop(0, n_chunks - NIDX, step=NIDX)
        def _(g0):
            for b in range(NIDX):
                g = g0 + b
                gather_wait(b % NBUF)
                scatter(b, b % NBUF)
                idx_start(g + NIDX, b)
                idx_wait((b + NBUF) % NIDX)
                gather_start(g + NBUF, (b + NBUF) % NIDX, b % NBUF)

        for b in range(NIDX):
            g = n_chunks - NIDX + b
            gather_wait(b % NBUF)
            scatter(b, b % NBUF)
            if b + NBUF < NIDX:
                idx_wait((b + NBUF) % NIDX)
                gather_start(g + NBUF, (b + NBUF) % NIDX, b % NBUF)
        plsc.subcore_barrier()

        pltpu.sync_copy(
            acc.at[pl.ds(row_base, rows_per_tile)],
            out_hbm.at[c, pl.ds(row_base, rows_per_tile)],
        )

    return sc_kernel(x, edges)


def _finalize(x, acc):
    n_nodes, d_feat = x.shape
    br = 1000

    def body(x_ref, acc_ref, o_ref):
        o_ref[:, :d_feat] = x_ref[...]
        o_ref[:, d_feat:] = acc_ref[0] + acc_ref[1]

    return pl.pallas_call(
        body,
        grid=(n_nodes // br,),
        in_specs=[
            pl.BlockSpec((br, d_feat), lambda i: (i, 0)),
            pl.BlockSpec((NC, br, d_feat), lambda i: (0, i, 0)),
        ],
        out_specs=pl.BlockSpec((br, 2 * d_feat), lambda i: (i, 0)),
        out_shape=jax.ShapeDtypeStruct((n_nodes, 2 * d_feat), jnp.float32),
    )(x, acc)


@jax.jit
def _run(x, edges):
    acc = _sc_scatter_add(x, edges)
    return _finalize(x, acc)


def kernel(x, edge_index, batch_size):
    n_nodes = x.shape[0]
    n_edges = edge_index.shape[1]
    ei = edge_index.astype(jnp.int32)
    n_padded = NS * (NCH0 + NCH1) * K
    pad = n_padded - n_edges
    src = jnp.concatenate([ei[0], jnp.zeros((pad,), jnp.int32)])
    dst = jnp.concatenate([ei[1], jnp.full((pad,), n_nodes, jnp.int32)])
    edges = jnp.stack(
        [src.reshape(-1, K), dst.reshape(-1, K)], axis=1
    )
    return _run(x, edges)
